# Initial kernel scaffold; baseline (speedup 1.0000x reference)
import jax
import jax.numpy as jnp
from jax.experimental import pallas as pl


def kernel(target_embeddings, context_embeddings, target_biases, context_biases, co_occurs, target_ind, context_ind):
    raise NotImplementedError("write your pallas kernel here")


# speedup vs baseline: 1.3721x; 1.3721x over previous
import functools

import jax
import jax.numpy as jnp
from jax import lax
from jax.experimental import pallas as pl
from jax.experimental.pallas import tpu as pltpu
from jax.experimental.pallas import tpu_sc as plsc

V = 100000
D = 128
B = 16384
L = 16
NC = 2
NS = 16
NW = NC * NS
BPW = B // NW
CH = 256
NCH = BPW // CH

_mesh = plsc.VectorSubcoreMesh(core_axis_name="c", subcore_axis_name="s")


@functools.partial(
    pl.kernel,
    mesh=_mesh,
    compiler_params=pltpu.CompilerParams(needs_layout_passes=False),
    out_type=jax.ShapeDtypeStruct((B,), jnp.float32),
    scratch_types=(
        [pltpu.VMEM((CH,), jnp.int32) for _ in range(2 * NCH)]
        + [pltpu.VMEM((CH,), jnp.float32) for _ in range(2 * NCH)]
        + [
            pltpu.VMEM((CH, D), jnp.float32),
            pltpu.VMEM((CH, D), jnp.float32),
            pltpu.VMEM((CH * L,), jnp.float32),
            pltpu.VMEM((BPW,), jnp.float32),
            pltpu.SemaphoreType.DMA,
        ]
    ),
)
def _sc_dot(t_emb, c_emb, t_bias, c_bias, t_ind, c_ind, out_hbm, *scratch):
    t_idx_v = scratch[0:NCH]
    c_idx_v = scratch[NCH:2 * NCH]
    t_bias_v = scratch[2 * NCH:3 * NCH]
    c_bias_v = scratch[3 * NCH:4 * NCH]
    t_rows, c_rows, pp_v, p_v, sem = scratch[4 * NCH:]
    wid = lax.axis_index("s") * NC + lax.axis_index("c")
    base = wid * BPW
    for ch in range(NCH):
        pltpu.sync_copy(t_ind.at[pl.ds(base + ch * CH, CH)], t_idx_v[ch])
        pltpu.sync_copy(c_ind.at[pl.ds(base + ch * CH, CH)], c_idx_v[ch])
    for ch in range(NCH):
        pltpu.async_copy(t_bias.at[t_idx_v[ch]], t_bias_v[ch], sem)
        pltpu.async_copy(c_bias.at[c_idx_v[ch]], c_bias_v[ch], sem)
        pltpu.async_copy(t_emb.at[t_idx_v[ch]], t_rows, sem)
        pltpu.async_copy(c_emb.at[c_idx_v[ch]], c_rows, sem)
        pltpu.make_async_copy(t_bias.at[t_idx_v[ch]], t_bias_v[ch], sem).wait()
        pltpu.make_async_copy(c_bias.at[c_idx_v[ch]], c_bias_v[ch], sem).wait()
        pltpu.make_async_copy(t_emb.at[t_idx_v[ch]], t_rows, sem).wait()
        pltpu.make_async_copy(c_emb.at[c_idx_v[ch]], c_rows, sem).wait()
        def elem_body(e, _):
            v = t_rows[e, pl.ds(0, L)] * c_rows[e, pl.ds(0, L)]
            for j in range(1, D // L):
                v = v + t_rows[e, pl.ds(j * L, L)] * c_rows[e, pl.ds(j * L, L)]
            pp_v[pl.ds(e * L, L)] = v
            return 0

        lax.fori_loop(0, CH, elem_body, 0)

        for g in range(CH // L):
            base_ids = g * L * L + lax.iota(jnp.int32, 16) * L

            def red_body(l, acc):
                return acc + plsc.load_gather(pp_v, [base_ids + l])

            acc = lax.fori_loop(0, L, red_body, jnp.zeros((16,), jnp.float32))
            p = acc + t_bias_v[ch][pl.ds(g * L, L)] + c_bias_v[ch][pl.ds(g * L, L)]
            p_v[pl.ds(ch * CH + g * L, L)] = p
    pltpu.sync_copy(p_v, out_hbm.at[pl.ds(base, BPW)])


def _tc_weighted_loss(p_ref, co_ref, out_ref):
    p = p_ref[...]
    co = co_ref[...]
    w = jnp.minimum(1.0, jnp.power(co * (1.0 / 100.0), 0.75))
    dist = p - jnp.log(co + 1.0)
    out_ref[...] = jnp.sum(w * dist * dist).reshape(1, 1)


def kernel(target_embeddings, context_embeddings, target_biases, context_biases,
           co_occurs, target_ind, context_ind):
    p = _sc_dot(target_embeddings, context_embeddings, target_biases,
                context_biases, target_ind, context_ind)
    out = pl.pallas_call(
        _tc_weighted_loss,
        out_shape=jax.ShapeDtypeStruct((1, 1), jnp.float32),
    )(p.reshape(128, 128), co_occurs.reshape(128, 128))
    return out[0, 0]
